# Initial kernel scaffold; baseline (speedup 1.0000x reference)
import jax
import jax.numpy as jnp
from jax.experimental import pallas as pl


def kernel(z, W):
    raise NotImplementedError("write your pallas kernel here")


# speedup vs baseline: 1.6295x; 1.6295x over previous
import jax
import jax.numpy as jnp
from jax.experimental import pallas as pl

_N_CODES = 1024
_DIM = 32
_COMMIT = 0.25


def _vq_body(z_ref, w_ref, out_ref, loss_ref):
    b = pl.program_id(0)
    zb = z_ref[0]
    w = w_ref[...]
    zbt = zb.T
    wsq = jnp.sum(w * w, axis=1)
    zsq = jnp.sum(zbt * zbt, axis=1)
    mm = jax.lax.dot_general(zbt, w, (((1,), (1,)), ((), ())),
                             preferred_element_type=jnp.float32)
    dists = (zsq[:, None] + wsq[None, :]) - 2.0 * mm
    dmin = jnp.min(dists, axis=1, keepdims=True)
    lane = jax.lax.broadcasted_iota(jnp.int32, dists.shape, 1)
    idx = jnp.min(jnp.where(dists == dmin, lane, _N_CODES), axis=1)
    oh = (lane == idx[:, None]).astype(jnp.float32)
    zq = jax.lax.dot_general(oh, w, (((1,), (0,)), ((), ())),
                             preferred_element_type=jnp.float32)
    zqt = zq.T
    diff = zqt - zb
    out_ref[0] = zb + diff

    @pl.when(b == 0)
    def _():
        loss_ref[...] = jnp.zeros_like(loss_ref)

    loss_ref[...] += jnp.sum(diff * diff, keepdims=True)


def kernel(z, W):
    B, C, H, Wd = z.shape
    HW = H * Wd
    z3 = z.reshape(B, C, HW)
    out, loss = pl.pallas_call(
        _vq_body,
        grid=(B,),
        in_specs=[
            pl.BlockSpec((1, C, HW), lambda b: (b, 0, 0)),
            pl.BlockSpec((_N_CODES, _DIM), lambda b: (0, 0)),
        ],
        out_specs=[
            pl.BlockSpec((1, C, HW), lambda b: (b, 0, 0)),
            pl.BlockSpec((1, 1), lambda b: (0, 0)),
        ],
        out_shape=[
            jax.ShapeDtypeStruct((B, C, HW), jnp.float32),
            jax.ShapeDtypeStruct((1, 1), jnp.float32),
        ],
    )(z3, W)
    m = loss[0, 0] / (B * C * H * Wd)
    vq_loss = m + _COMMIT * m
    return out.reshape(B, C, H, Wd), vq_loss
